# baseline (speedup 1.0000x reference)
import jax
import jax.numpy as jnp
from jax import lax
from jax.experimental import pallas as pl
from jax.experimental.pallas import tpu as pltpu
from jax.experimental.pallas import tpu_sc as plsc

N_CLASSES = 1000
N_REGIMES = 50
N_EXCIPIENTS = 200
B = 16384

NC = 2
NS = 16
L = 16
NW = NC * NS
BPW = B // NW

_LOG1P_COEFFS = (
    3.5960209743279847e-07, 0.9999747275494544, -0.49955556043099913,
    0.32988771549753426, -0.23519411941265345, 0.1599719210060835,
    -0.09237501315110101, 0.040762090744950866, -0.012419734085963167,
    0.002283966446104766, -0.000189158643389574)


def _log1p(x):
    acc = jnp.full((L,), _LOG1P_COEFFS[-1], jnp.float32)
    for c in _LOG1P_COEFFS[-2::-1]:
        acc = acc * x + c
    return acc


def _body(p_hbm, r_hbm, e_hbm, conc_hbm, sd_hbm, thr_hbm, ww_hbm,
          res_hbm, gate_hbm, ct_hbm,
          p_v, r_v, e_v, conc_v, thr_v, idx_v, idx2_v,
          sd_v, ww_v, res_v, gate_v, ct_v, sem):
    wid = lax.axis_index("s") * NC + lax.axis_index("c")
    base = wid * BPW

    pltpu.sync_copy(p_hbm.at[pl.ds(base, BPW)], p_v)
    pltpu.sync_copy(r_hbm.at[pl.ds(base, BPW)], r_v)
    pltpu.sync_copy(e_hbm.at[pl.ds(base, BPW)], e_v)
    pltpu.sync_copy(conc_hbm.at[pl.ds(base, BPW)], conc_v)
    pltpu.sync_copy(thr_hbm, thr_v)

    for i in range(BPW // L):
        sl = pl.ds(i * L, L)
        idx_v[sl] = (p_v[sl] * (N_REGIMES * N_EXCIPIENTS)
                     + r_v[sl] * N_EXCIPIENTS + e_v[sl])

    for i in range(BPW // L):
        sl = pl.ds(i * L, L)
        idx2_v[sl] = lax.shift_right_logical(idx_v[sl], jnp.int32(1))

    h1 = pltpu.async_copy(sd_hbm.at[idx_v], sd_v, sem)
    h2 = pltpu.async_copy(ww_hbm.at[idx2_v], ww_v, sem)
    h1.wait()
    h2.wait()

    for i in range(BPW // L):
        sl = pl.ds(i * L, L)
        thr = plsc.load_gather(thr_v, [e_v[sl]])
        cr = conc_v[sl] / (thr + 1e-6)
        gate = 1.0 / (1.0 + jnp.exp(-10.0 * (cr - 1.0)))
        tanh_cr = 1.0 - 2.0 / (jnp.exp(2.0 * cr) + 1.0)
        lg = _log1p(cr)
        w = lax.bitcast_convert_type(sd_v[sl], jnp.int32)
        sd = lax.bitcast_convert_type(
            lax.bitwise_and(w, jnp.int32(-65536)), jnp.float32)
        w_b = lax.bitcast_convert_type(
            lax.shift_left(w, jnp.int32(16)), jnp.float32)
        w2 = lax.bitcast_convert_type(ww_v[sl], jnp.int32)
        odd = lax.bitwise_and(idx_v[sl], jnp.int32(1)) == 1
        w_a = lax.bitcast_convert_type(
            jnp.where(odd, lax.bitwise_and(w2, jnp.int32(-65536)),
                      lax.shift_left(w2, jnp.int32(16))), jnp.float32)
        ct = (1.0 - gate) * (tanh_cr * w_b) + gate * (lg * w_a)
        res_v[sl] = sd * ct
        gate_v[sl] = gate
        ct_v[sl] = ct

    pltpu.sync_copy(res_v, res_hbm.at[pl.ds(base, BPW)])
    pltpu.sync_copy(gate_v, gate_hbm.at[pl.ds(base, BPW)])
    pltpu.sync_copy(ct_v, ct_hbm.at[pl.ds(base, BPW)])


@jax.jit
def _run(p_idx, r_idx, e_idx, conc, sd, thr, ww):
    f32 = jnp.float32
    k = pl.kernel(
        _body,
        mesh=plsc.VectorSubcoreMesh(core_axis_name="c", subcore_axis_name="s"),
        out_type=[jax.ShapeDtypeStruct((B,), f32)] * 3,
        scratch_types=(
            [pltpu.VMEM((BPW,), jnp.int32)] * 3
            + [pltpu.VMEM((BPW,), f32)]
            + [pltpu.VMEM((N_EXCIPIENTS,), f32)]
            + [pltpu.VMEM((BPW,), jnp.int32)] * 2
            + [pltpu.VMEM((BPW,), f32)]
            + [pltpu.VMEM((BPW,), f32)]
            + [pltpu.VMEM((BPW,), f32)] * 3
            + [pltpu.SemaphoreType.DMA]
        ),
        compiler_params=pltpu.CompilerParams(needs_layout_passes=False),
    )
    return k(p_idx, r_idx, e_idx, conc, sd, thr, ww)


def kernel(p_idx, r_idx, e_idx, raw_concentration, static_scores, delta,
           thresholds, w_below, w_above):
    i32 = jnp.int32
    one = thresholds[0] * 0.0 + 1.0
    sd16 = lax.bitcast_convert_type(
        ((static_scores + jnp.clip(delta, -2.0, 2.0)) * one)
        .astype(jnp.bfloat16), jnp.uint16).astype(jnp.uint32)
    wb16 = lax.bitcast_convert_type(
        (w_below * one).astype(jnp.bfloat16), jnp.uint16).astype(jnp.uint32)
    sdwb = lax.bitcast_convert_type(
        lax.bitwise_or(lax.shift_left(sd16, jnp.uint32(16)), wb16),
        jnp.float32).reshape(-1)
    wa_lo = lax.bitcast_convert_type(
        (w_above[..., 0::2] * one).astype(jnp.bfloat16),
        jnp.uint16).astype(jnp.uint32)
    wa_hi = lax.bitcast_convert_type(
        (w_above[..., 1::2] * one).astype(jnp.bfloat16),
        jnp.uint16).astype(jnp.uint32)
    wa_pairs = lax.bitcast_convert_type(
        lax.bitwise_or(lax.shift_left(wa_hi, jnp.uint32(16)), wa_lo),
        jnp.float32).reshape(-1)
    res, gate, ct = _run(
        p_idx.astype(i32), r_idx.astype(i32), e_idx.astype(i32),
        raw_concentration.reshape(-1).astype(jnp.float32),
        sdwb, thresholds, wa_pairs)
    return (res[:, None], gate[:, None], ct[:, None])


# speedup vs baseline: 2.0636x; 2.0636x over previous
import jax
import jax.numpy as jnp
from jax import lax
from jax.experimental import pallas as pl
from jax.experimental.pallas import tpu as pltpu
from jax.experimental.pallas import tpu_sc as plsc

N_CLASSES = 1000
N_REGIMES = 50
N_EXCIPIENTS = 200
B = 16384

NC = 2
NS = 16
L = 16
NW = NC * NS
BPW = B // NW

_LOG1P_COEFFS = (
    3.5960209743279847e-07, 0.9999747275494544, -0.49955556043099913,
    0.32988771549753426, -0.23519411941265345, 0.1599719210060835,
    -0.09237501315110101, 0.040762090744950866, -0.012419734085963167,
    0.002283966446104766, -0.000189158643389574)


def _log1p(x):
    acc = jnp.full((L,), _LOG1P_COEFFS[-1], jnp.float32)
    for c in _LOG1P_COEFFS[-2::-1]:
        acc = acc * x + c
    return acc


def _body(p_hbm, r_hbm, e_hbm, conc_hbm, sd_hbm, thr_hbm, ww_hbm,
          res_hbm, gate_hbm, ct_hbm,
          p_v, r_v, e_v, conc_v, thr_v, idx_v,
          sd_v, ww_v, res_v, gate_v, ct_v, sem):
    wid = lax.axis_index("s") * NC + lax.axis_index("c")
    base = wid * BPW

    pltpu.sync_copy(p_hbm.at[pl.ds(base, BPW)], p_v)
    pltpu.sync_copy(r_hbm.at[pl.ds(base, BPW)], r_v)
    pltpu.sync_copy(e_hbm.at[pl.ds(base, BPW)], e_v)
    pltpu.sync_copy(conc_hbm.at[pl.ds(base, BPW)], conc_v)
    pltpu.sync_copy(thr_hbm, thr_v)

    for i in range(BPW // L):
        sl = pl.ds(i * L, L)
        idx_v[sl] = (p_v[sl] * (N_REGIMES * N_EXCIPIENTS)
                     + r_v[sl] * N_EXCIPIENTS + e_v[sl])

    h1 = pltpu.async_copy(sd_hbm.at[idx_v], sd_v, sem)
    h2 = pltpu.async_copy(ww_hbm.at[idx_v], ww_v, sem)
    h1.wait()
    h2.wait()

    for i in range(BPW // L):
        sl = pl.ds(i * L, L)
        thr = plsc.load_gather(thr_v, [e_v[sl]])
        cr = conc_v[sl] / (thr + 1e-6)
        gate = 1.0 / (1.0 + jnp.exp(-10.0 * (cr - 1.0)))
        tanh_cr = 1.0 - 2.0 / (jnp.exp(2.0 * cr) + 1.0)
        lg = _log1p(cr)
        w = lax.bitcast_convert_type(ww_v[sl], jnp.int32)
        w_b = lax.bitcast_convert_type(
            lax.bitwise_and(w, jnp.int32(-65536)), jnp.float32)
        w_a = lax.bitcast_convert_type(
            lax.shift_left(w, jnp.int32(16)), jnp.float32)
        ct = (1.0 - gate) * (tanh_cr * w_b) + gate * (lg * w_a)
        res_v[sl] = sd_v[sl] * ct
        gate_v[sl] = gate
        ct_v[sl] = ct

    pltpu.sync_copy(res_v, res_hbm.at[pl.ds(base, BPW)])
    pltpu.sync_copy(gate_v, gate_hbm.at[pl.ds(base, BPW)])
    pltpu.sync_copy(ct_v, ct_hbm.at[pl.ds(base, BPW)])


@jax.jit
def _run(p_idx, r_idx, e_idx, conc, sd, thr, ww):
    f32 = jnp.float32
    k = pl.kernel(
        _body,
        mesh=plsc.VectorSubcoreMesh(core_axis_name="c", subcore_axis_name="s"),
        out_type=[jax.ShapeDtypeStruct((B,), f32)] * 3,
        scratch_types=(
            [pltpu.VMEM((BPW,), jnp.int32)] * 3
            + [pltpu.VMEM((BPW,), f32)]
            + [pltpu.VMEM((N_EXCIPIENTS,), f32)]
            + [pltpu.VMEM((BPW,), jnp.int32)]
            + [pltpu.VMEM((BPW,), f32)]
            + [pltpu.VMEM((BPW,), f32)]
            + [pltpu.VMEM((BPW,), f32)] * 3
            + [pltpu.SemaphoreType.DMA]
        ),
        compiler_params=pltpu.CompilerParams(needs_layout_passes=False),
    )
    return k(p_idx, r_idx, e_idx, conc, sd, thr, ww)


def kernel(p_idx, r_idx, e_idx, raw_concentration, static_scores, delta,
           thresholds, w_below, w_above):
    i32 = jnp.int32
    one = thresholds[0] * 0.0 + 1.0
    sd = ((static_scores + jnp.clip(delta, -2.0, 2.0)) * one).reshape(-1)
    wb16 = lax.bitcast_convert_type(
        (w_below * one).astype(jnp.bfloat16), jnp.uint16).astype(jnp.uint32)
    wa16 = lax.bitcast_convert_type(
        (w_above * one).astype(jnp.bfloat16), jnp.uint16).astype(jnp.uint32)
    ww = lax.bitcast_convert_type(
        lax.bitwise_or(lax.shift_left(wb16, jnp.uint32(16)), wa16),
        jnp.float32).reshape(-1)
    res, gate, ct = _run(
        p_idx.astype(i32), r_idx.astype(i32), e_idx.astype(i32),
        raw_concentration.reshape(-1).astype(jnp.float32),
        sd, thresholds, ww)
    return (res[:, None], gate[:, None], ct[:, None])
